# baseline (speedup 1.0000x reference)
import functools

import jax
import jax.numpy as jnp
from jax import lax
from jax.experimental import pallas as pl
from jax.experimental.pallas import tpu as pltpu
from jax.experimental.pallas import tpu_sc as plsc

_B = 4
_S = 4096
_D = 1024
_N = _B * _S
_NC = 2
_NS = 16
_NW = _NC * _NS
_RW = _N // _NW
_C = 32
_NCHUNK = _RW // _C
_NB = 3
_L = 16


def _sc_kernel(tok_hbm, w_hbm, out_hbm, tok_v, idx_v, buf0, buf1, buf2,
               gsem0, gsem1, gsem2, osem0, osem1, osem2):
    wid = lax.axis_index("s") * _NC + lax.axis_index("c")
    base = wid * _RW
    pos0 = lax.rem(base, _S) + 1

    pltpu.sync_copy(tok_hbm.at[pl.ds(base, _RW)], tok_v)

    def build_idx(j, _):
        tok = tok_v[pl.ds(j * _L, _L)]
        ramp = lax.iota(jnp.int32, _L) + (pos0 + j * _L)
        idx_v[pl.ds(j * _L, _L)] = jnp.where(tok != 0, ramp, 0)
        return 0

    lax.fori_loop(0, _RW // _L, build_idx, 0)

    bufs = (buf0, buf1, buf2)
    gsems = (gsem0, gsem1, gsem2)
    osems = (osem0, osem1, osem2)

    def gather(c):
        p = c % _NB
        return pltpu.async_copy(w_hbm.at[idx_v.at[pl.ds(c * _C, _C)]],
                                bufs[p], gsems[p])

    pending = [gather(c) for c in range(_NB)]
    writes = [None] * _NCHUNK
    for c in range(_NCHUNK):
        p = c % _NB
        pending[p].wait()
        writes[c] = pltpu.async_copy(
            bufs[p], out_hbm.at[pl.ds(base + c * _C, _C)], osems[p])
        if c >= 1 and c + 2 < _NCHUNK:
            writes[c - 1].wait()
            writes[c - 1] = None
            pending[(c + 2) % _NB] = gather(c + 2)
    for wcp in writes:
        if wcp is not None:
            wcp.wait()


@jax.jit
def _run(tok_flat, weights):
    mesh = plsc.VectorSubcoreMesh(core_axis_name="c", subcore_axis_name="s")
    f = functools.partial(
        pl.kernel,
        mesh=mesh,
        compiler_params=pltpu.CompilerParams(use_tc_tiling_on_sc=True),
        out_type=jax.ShapeDtypeStruct((_N, _D), jnp.float32),
        scratch_types=[
            pltpu.VMEM((_RW,), jnp.int32),
            pltpu.VMEM((_RW,), jnp.int32),
            pltpu.VMEM((_C, _D), jnp.float32),
            pltpu.VMEM((_C, _D), jnp.float32),
            pltpu.VMEM((_C, _D), jnp.float32),
            pltpu.SemaphoreType.DMA,
            pltpu.SemaphoreType.DMA,
            pltpu.SemaphoreType.DMA,
            pltpu.SemaphoreType.DMA,
            pltpu.SemaphoreType.DMA,
            pltpu.SemaphoreType.DMA,
        ],
    )(_sc_kernel)
    return f(tok_flat, weights)


def kernel(input, weights):
    tok_flat = input.reshape(-1)
    out = _run(tok_flat, weights)
    return out.reshape(_B, _S, _D)


# speedup vs baseline: 1.0002x; 1.0002x over previous
import functools

import jax
import jax.numpy as jnp
from jax import lax
from jax.experimental import pallas as pl
from jax.experimental.pallas import tpu as pltpu
from jax.experimental.pallas import tpu_sc as plsc

_B = 4
_S = 4096
_D = 1024
_N = _B * _S
_NC = 2
_NS = 16
_NW = _NC * _NS
_RW = _N // _NW
_C = 32
_NCHUNK = _RW // _C
_NB = 3
_L = 16


def _sc_kernel(tok_hbm, w_hbm, out_hbm, tok_v, idx_v, buf0, buf1, buf2,
               gsem0, gsem1, gsem2, osem0, osem1, osem2):
    wid = lax.axis_index("s") * _NC + lax.axis_index("c")
    base = wid * _RW
    pos0 = lax.rem(base, _S) + 1

    pltpu.sync_copy(tok_hbm.at[pl.ds(base, _RW)], tok_v)

    def build_idx(j, _):
        tok = tok_v[pl.ds(j * _L, _L)]
        ramp = lax.iota(jnp.int32, _L) + (pos0 + j * _L)
        idx_v[pl.ds(j * _L, _L)] = jnp.where(tok != 0, ramp, 0)
        return 0

    lax.fori_loop(0, _RW // _L, build_idx, 0)

    bufs = (buf0, buf1, buf2)
    gsems = (gsem0, gsem1, gsem2)
    osems = (osem0, osem1, osem2)

    def gather(c):
        p = c % _NB
        return pltpu.async_copy(w_hbm.at[idx_v.at[pl.ds(c * _C, _C)]],
                                bufs[p], gsems[p])

    pending = [gather(c) for c in range(_NB)]
    writes = [None] * _NCHUNK
    for c in range(_NCHUNK):
        p = c % _NB
        pending[p].wait()
        writes[c] = pltpu.async_copy(
            bufs[p], out_hbm.at[pl.ds(base + c * _C, _C)], osems[p])
        if c >= 1 and c + 2 < _NCHUNK:
            writes[c - 1].wait()
            writes[c - 1] = None
            pending[(c + 2) % _NB] = gather(c + 2)
    for wcp in writes:
        if wcp is not None:
            wcp.wait()


@jax.jit
def _run(tok_flat, weights):
    mesh = plsc.VectorSubcoreMesh(core_axis_name="c", subcore_axis_name="s")
    f = functools.partial(
        pl.kernel,
        mesh=mesh,
        compiler_params=pltpu.CompilerParams(use_tc_tiling_on_sc=True, disable_bounds_checks=True, disable_semaphore_checks=True),
        out_type=jax.ShapeDtypeStruct((_N, _D), jnp.float32),
        scratch_types=[
            pltpu.VMEM((_RW,), jnp.int32),
            pltpu.VMEM((_RW,), jnp.int32),
            pltpu.VMEM((_C, _D), jnp.float32),
            pltpu.VMEM((_C, _D), jnp.float32),
            pltpu.VMEM((_C, _D), jnp.float32),
            pltpu.SemaphoreType.DMA,
            pltpu.SemaphoreType.DMA,
            pltpu.SemaphoreType.DMA,
            pltpu.SemaphoreType.DMA,
            pltpu.SemaphoreType.DMA,
            pltpu.SemaphoreType.DMA,
        ],
    )(_sc_kernel)
    return f(tok_flat, weights)


def kernel(input, weights):
    tok_flat = input.reshape(-1)
    out = _run(tok_flat, weights)
    return out.reshape(_B, _S, _D)


# speedup vs baseline: 1.0063x; 1.0061x over previous
import functools

import jax
import jax.numpy as jnp
from jax import lax
from jax.experimental import pallas as pl
from jax.experimental.pallas import tpu as pltpu
from jax.experimental.pallas import tpu_sc as plsc

_B = 4
_S = 4096
_D = 1024
_N = _B * _S
_NC = 2
_NS = 16
_NW = _NC * _NS
_RW = _N // _NW
_C = 32
_NCHUNK = _RW // _C
_NB = 3
_L = 16


def _sc_kernel(tok_hbm, w_hbm, out_hbm, tok_v, idx_v, buf0, buf1, buf2,
               gsem0, gsem1, gsem2, osem0, osem1, osem2):
    wid = lax.axis_index("s") * _NC + lax.axis_index("c")
    base = wid * _RW
    pos0 = lax.rem(base, _S) + 1

    pltpu.sync_copy(tok_hbm.at[pl.ds(base, _RW)], tok_v)

    def build_idx(j, _):
        tok = tok_v[pl.ds(j * _L, _L)]
        ramp = lax.iota(jnp.int32, _L) + (pos0 + j * _L)
        idx_v[pl.ds(j * _L, _L)] = jnp.where(tok != 0, ramp, 0)
        return 0

    lax.fori_loop(0, _RW // _L, build_idx, 0)

    bufs = (buf0, buf1, buf2)
    gsems = (gsem0, gsem1, gsem2)
    osems = (osem0, osem1, osem2)

    def gather(c):
        p = c % _NB
        return pltpu.async_copy(w_hbm.at[idx_v.at[pl.ds(c * _C, _C)]],
                                bufs[p], gsems[p])

    pending = [gather(c) for c in range(_NB)]
    writes = [None] * _NCHUNK
    for c in range(_NCHUNK):
        p = c % _NB
        pending[p].wait()
        writes[c] = pltpu.async_copy(
            bufs[p], out_hbm.at[pl.ds(base + c * _C, _C)], osems[p])
        if c >= 1 and c + 2 < _NCHUNK:
            writes[c - 1].wait()
            writes[c - 1] = None
            pending[(c + 2) % _NB] = gather(c + 2)
    for wcp in writes:
        if wcp is not None:
            wcp.wait()


@jax.jit
def _run(tok_flat, weights):
    mesh = plsc.VectorSubcoreMesh(core_axis_name="c", subcore_axis_name="s")
    f = functools.partial(
        pl.kernel,
        mesh=mesh,
        compiler_params=pltpu.CompilerParams(use_tc_tiling_on_sc=True),
        out_type=jax.ShapeDtypeStruct((_N, _D), jnp.float32),
        scratch_types=[
            pltpu.VMEM((_RW,), jnp.int32),
            pltpu.VMEM((_RW,), jnp.int32),
            pltpu.VMEM((_C, _D), jnp.float32),
            pltpu.VMEM((_C, _D), jnp.float32),
            pltpu.VMEM((_C, _D), jnp.float32),
            pltpu.SemaphoreType.DMA,
            pltpu.SemaphoreType.DMA,
            pltpu.SemaphoreType.DMA,
            pltpu.SemaphoreType.DMA,
            pltpu.SemaphoreType.DMA,
            pltpu.SemaphoreType.DMA,
        ],
    )(_sc_kernel)
    return f(tok_flat, weights)


def kernel(input, weights):
    tok_flat = input.reshape(-1)
    out = _run(tok_flat, weights)
    return out.reshape(_B, _S, _D)


# speedup vs baseline: 1.0081x; 1.0018x over previous
import functools

import jax
import jax.numpy as jnp
from jax import lax
from jax.experimental import pallas as pl
from jax.experimental.pallas import tpu as pltpu
from jax.experimental.pallas import tpu_sc as plsc

_B = 4
_S = 4096
_D = 1024
_N = _B * _S
_NC = 2
_NS = 16
_NW = _NC * _NS
_RW = _N // _NW
_C = 56
_L = 16

_CHUNKS = [(i * _C, _C) for i in range(_RW // _C)]
_CHUNKS.append(((_RW // _C) * _C, _RW - (_RW // _C) * _C))


def _sc_kernel(tok_hbm, w_hbm, out_hbm, tok_v, idx_v, buf0, buf1, gsem0,
               gsem1, osem0, osem1):
    wid = lax.axis_index("s") * _NC + lax.axis_index("c")
    base = wid * _RW
    pos0 = lax.rem(base, _S) + 1

    pltpu.sync_copy(tok_hbm.at[pl.ds(base, _RW)], tok_v)

    def build_idx(j, _):
        tok = tok_v[pl.ds(j * _L, _L)]
        ramp = lax.iota(jnp.int32, _L) + (pos0 + j * _L)
        idx_v[pl.ds(j * _L, _L)] = jnp.where(tok != 0, ramp, 0)
        return 0

    lax.fori_loop(0, _RW // _L, build_idx, 0)

    bufs = (buf0, buf1)
    gsems = (gsem0, gsem1)
    osems = (osem0, osem1)

    def gather(c):
        off, n = _CHUNKS[c]
        p = c % 2
        return pltpu.async_copy(w_hbm.at[idx_v.at[pl.ds(off, n)]],
                                bufs[p].at[pl.ds(0, n)], gsems[p])

    nchunk = len(_CHUNKS)
    pending = [gather(0), gather(1)]
    out_pending = [None, None]
    for c in range(nchunk):
        off, n = _CHUNKS[c]
        p = c % 2
        pending[p].wait()
        out_pending[p] = pltpu.async_copy(
            bufs[p].at[pl.ds(0, n)], out_hbm.at[pl.ds(base + off, n)],
            osems[p])
        if c + 2 < nchunk:
            out_pending[p].wait()
            pending[p] = gather(c + 2)
    out_pending[0].wait()
    out_pending[1].wait()


@jax.jit
def _run(tok_flat, weights):
    mesh = plsc.VectorSubcoreMesh(core_axis_name="c", subcore_axis_name="s")
    f = functools.partial(
        pl.kernel,
        mesh=mesh,
        compiler_params=pltpu.CompilerParams(use_tc_tiling_on_sc=True),
        out_type=jax.ShapeDtypeStruct((_N, _D), jnp.float32),
        scratch_types=[
            pltpu.VMEM((_RW,), jnp.int32),
            pltpu.VMEM((_RW,), jnp.int32),
            pltpu.VMEM((_C, _D), jnp.float32),
            pltpu.VMEM((_C, _D), jnp.float32),
            pltpu.SemaphoreType.DMA,
            pltpu.SemaphoreType.DMA,
            pltpu.SemaphoreType.DMA,
            pltpu.SemaphoreType.DMA,
        ],
    )(_sc_kernel)
    return f(tok_flat, weights)


def kernel(input, weights):
    tok_flat = input.reshape(-1)
    out = _run(tok_flat, weights)
    return out.reshape(_B, _S, _D)
